# baseline (speedup 1.0000x reference)
import functools
import math

import jax
import jax.numpy as jnp
from jax import lax
from jax.experimental import pallas as pl
from jax.experimental.pallas import tpu as pltpu
from jax.experimental.pallas import tpu_sc as plsc

D_MODEL = 64
SCALE = math.sqrt(D_MODEL)
B_TOTAL = 4096 * 200
NC, NS, L = 2, 16, 16
NW = NC * NS
CHUNK = 128
B_PER_W = B_TOTAL // NW
G_PER_W = B_PER_W // CHUNK

_mesh = plsc.VectorSubcoreMesh(core_axis_name="c", subcore_axis_name="s")


@functools.partial(
    pl.kernel,
    out_type=jax.ShapeDtypeStruct((B_TOTAL, D_MODEL), jnp.float32),
    mesh=_mesh,
    scratch_types=[
        pltpu.VMEM((G_PER_W, CHUNK), jnp.int32),
        pltpu.VMEM((CHUNK, D_MODEL), jnp.float32),
        pltpu.SemaphoreType.DMA,
    ],
    compiler_params=pltpu.CompilerParams(use_tc_tiling_on_sc=False),
)
def _gather_scale(x_hbm, table_hbm, out_hbm, idx_v, rows_v, sem):
    wid = lax.axis_index("s") * NC + lax.axis_index("c")
    pltpu.sync_copy(x_hbm.at[pl.ds(wid * G_PER_W, G_PER_W)], idx_v)
    out_base = wid * B_PER_W

    def chunk_body(g, carry):
        pltpu.async_copy(table_hbm.at[idx_v.at[g]], rows_v, sem).wait()

        def scale_row(r, c2):
            for j in range(D_MODEL // L):
                sl = pl.ds(j * L, L)
                rows_v[r, sl] = rows_v[r, sl] * SCALE
            return c2

        lax.fori_loop(0, CHUNK, scale_row, 0)
        pltpu.sync_copy(rows_v, out_hbm.at[pl.ds(out_base + g * CHUNK, CHUNK)])
        return carry

    lax.fori_loop(0, G_PER_W, chunk_body, 0)


def kernel(x, table):
    xf = x.reshape(-1).astype(jnp.int32).reshape(B_TOTAL // CHUNK, CHUNK)
    out = _gather_scale(xf, table)
    return out.reshape(x.shape + (D_MODEL,))


# speedup vs baseline: 1.2079x; 1.2079x over previous
import functools
import math

import jax
import jax.numpy as jnp
from jax import lax
from jax.experimental import pallas as pl
from jax.experimental.pallas import tpu as pltpu
from jax.experimental.pallas import tpu_sc as plsc

D_MODEL = 64
SCALE = math.sqrt(D_MODEL)
B_TOTAL = 4096 * 200
NC, NS, L = 2, 16, 16
NW = NC * NS
CHUNK = 128
K = 4
SUP = K * CHUNK
B_PER_W = B_TOTAL // NW
G_PER_W = B_PER_W // CHUNK
NSUP = G_PER_W // K

_mesh = plsc.VectorSubcoreMesh(core_axis_name="c", subcore_axis_name="s")


@functools.partial(
    pl.kernel,
    out_type=jax.ShapeDtypeStruct((B_TOTAL, D_MODEL), jnp.float32),
    mesh=_mesh,
    scratch_types=[
        pltpu.VMEM((G_PER_W, CHUNK), jnp.int32),
        pltpu.VMEM((SUP, D_MODEL), jnp.float32),
        pltpu.VMEM((SUP, D_MODEL), jnp.float32),
        pltpu.SemaphoreType.DMA,
        pltpu.SemaphoreType.DMA,
        pltpu.SemaphoreType.DMA,
    ],
    compiler_params=pltpu.CompilerParams(use_tc_tiling_on_sc=False),
)
def _gather_scale(x_hbm, table_hbm, out_hbm, idx_v, big0, big1, gsem, ssem0, ssem1):
    wid = lax.axis_index("s") * NC + lax.axis_index("c")
    pltpu.sync_copy(x_hbm.at[pl.ds(wid * G_PER_W, G_PER_W)], idx_v)
    out_base = wid * B_PER_W

    def fire_gathers(s, buf):
        for j in range(K):
            pltpu.async_copy(
                table_hbm.at[idx_v.at[s * K + j]],
                buf.at[pl.ds(j * CHUNK, CHUNK)],
                gsem,
            )

    def drain_gathers(buf):
        pltpu.make_async_copy(out_hbm.at[pl.ds(out_base, SUP)], buf, gsem).wait()

    def scale(buf):
        @plsc.parallel_loop(0, SUP, unroll=8)
        def _(r):
            for j in range(D_MODEL // L):
                sl = pl.ds(j * L, L)
                buf[r, sl] = buf[r, sl] * SCALE

    def fire_store(s, buf, ssem):
        pltpu.async_copy(buf, out_hbm.at[pl.ds(out_base + s * SUP, SUP)], ssem)

    def drain_store(buf, ssem):
        pltpu.make_async_copy(buf, out_hbm.at[pl.ds(out_base, SUP)], ssem).wait()

    fire_gathers(0, big0)

    def super2(t, carry):
        s0 = 2 * t
        drain_gathers(big0)

        @pl.when(t > 0)
        def _():
            drain_store(big1, ssem1)

        fire_gathers(s0 + 1, big1)
        scale(big0)
        fire_store(s0, big0, ssem0)
        drain_gathers(big1)
        drain_store(big0, ssem0)

        @pl.when(t < NSUP // 2 - 1)
        def _():
            fire_gathers(s0 + 2, big0)

        scale(big1)
        fire_store(s0 + 1, big1, ssem1)
        return carry

    lax.fori_loop(0, NSUP // 2, super2, 0)
    drain_store(big1, ssem1)


def kernel(x, table):
    xf = x.reshape(-1).astype(jnp.int32).reshape(B_TOTAL // CHUNK, CHUNK)
    out = _gather_scale(xf, table)
    return out.reshape(x.shape + (D_MODEL,))


# speedup vs baseline: 1.5295x; 1.2663x over previous
import functools
import math

import jax
import jax.numpy as jnp
from jax import lax
from jax.experimental import pallas as pl
from jax.experimental.pallas import tpu as pltpu
from jax.experimental.pallas import tpu_sc as plsc

D_MODEL = 64
SCALE = math.sqrt(D_MODEL)
B_TOTAL = 4096 * 200
NC, NS, L = 2, 16, 16
NW = NC * NS
CHUNK = 128
K = 4
SUP = K * CHUNK
B_PER_W = B_TOTAL // NW
G_PER_W = B_PER_W // CHUNK
NSUP = G_PER_W // K

COLS = 2048
NBLK = 245
LROWS = NBLK * COLS

_mesh = plsc.VectorSubcoreMesh(core_axis_name="c", subcore_axis_name="s")


def _k1_body(a_ref, b_ref, o_ref):
    o_ref[:, 0:64] = a_ref[...].T * SCALE
    o_ref[:, 64:128] = b_ref[...].T * SCALE


def _linearize_table(tt):
    return pl.pallas_call(
        _k1_body,
        grid=(NBLK,),
        in_specs=[
            pl.BlockSpec((64, COLS), lambda i: (0, 2 * i)),
            pl.BlockSpec((64, COLS), lambda i: (0, jnp.minimum(2 * i + 1, 487))),
        ],
        out_specs=pl.BlockSpec((COLS, 128), lambda i: (i, 0)),
        out_shape=jax.ShapeDtypeStruct((LROWS, 128), jnp.float32),
    )(tt, tt)


@functools.partial(
    pl.kernel,
    out_type=jax.ShapeDtypeStruct((B_TOTAL, D_MODEL), jnp.float32),
    mesh=_mesh,
    scratch_types=[
        pltpu.VMEM((G_PER_W, CHUNK), jnp.int32),
        pltpu.VMEM((SUP, D_MODEL), jnp.float32),
        pltpu.VMEM((SUP, D_MODEL), jnp.float32),
        pltpu.SemaphoreType.DMA,
        pltpu.SemaphoreType.DMA,
        pltpu.SemaphoreType.DMA,
    ],
    compiler_params=pltpu.CompilerParams(use_tc_tiling_on_sc=False),
)
def _gather_rows(x_hbm, table_hbm, out_hbm, idx_v, big0, big1, gsem, ssem0, ssem1):
    wid = lax.axis_index("s") * NC + lax.axis_index("c")
    pltpu.sync_copy(x_hbm.at[pl.ds(wid * G_PER_W, G_PER_W)], idx_v)
    out_base = wid * B_PER_W

    def xf_row(t, carry):
        for j in range(CHUNK // L):
            sl = pl.ds(j * L, L)
            q = idx_v[t, sl]
            idx_v[t, sl] = ((q >> 12) << 12) + ((q & 2047) << 1) + ((q >> 11) & 1)
        return carry

    lax.fori_loop(0, G_PER_W, xf_row, 0)

    def fire_gathers(s, buf):
        for j in range(K):
            pltpu.async_copy(
                table_hbm.at[idx_v.at[s * K + j]],
                buf.at[pl.ds(j * CHUNK, CHUNK)],
                gsem,
            )

    def drain_gathers(buf):
        pltpu.make_async_copy(out_hbm.at[pl.ds(out_base, SUP)], buf, gsem).wait()

    def fire_store(s, buf, ssem):
        pltpu.async_copy(buf, out_hbm.at[pl.ds(out_base + s * SUP, SUP)], ssem)

    def drain_store(buf, ssem):
        pltpu.make_async_copy(buf, out_hbm.at[pl.ds(out_base, SUP)], ssem).wait()

    fire_gathers(0, big0)

    def super2(t, carry):
        s0 = 2 * t
        drain_gathers(big0)

        @pl.when(t > 0)
        def _():
            drain_store(big1, ssem1)

        fire_gathers(s0 + 1, big1)
        fire_store(s0, big0, ssem0)
        drain_gathers(big1)
        drain_store(big0, ssem0)

        @pl.when(t < NSUP // 2 - 1)
        def _():
            fire_gathers(s0 + 2, big0)

        fire_store(s0 + 1, big1, ssem1)
        return carry

    lax.fori_loop(0, NSUP // 2, super2, 0)
    drain_store(big1, ssem1)


def kernel(x, table):
    tt = table.T
    ltab = _linearize_table(tt).reshape(2 * LROWS, D_MODEL)
    xf = x.reshape(-1).astype(jnp.int32).reshape(B_TOTAL // CHUNK, CHUNK)
    out = _gather_rows(xf, ltab)
    return out.reshape(x.shape + (D_MODEL,))
